# baseline (speedup 1.0000x reference)
import functools

import jax
import jax.numpy as jnp
from jax import lax
from jax.experimental import pallas as pl
from jax.experimental.pallas import tpu as pltpu
from jax.experimental.pallas import tpu_sc as plsc

N = 50000
E = 800000
H = 64

NC = 2
NS = 16
NW = NC * NS
LANES = 16

ROWS = 392
N_PAD = ROWS * 128

EPT = E // NW
CHUNK = 5000
NCHUNK = EPT // CHUNK
NVEC = CHUNK // LANES
TAIL = CHUNK - NVEC * LANES
UNROLL = 4
BUF = NVEC * LANES + LANES

HALF = N_PAD // 2
DUM = 16
ACC_ROWS = HALF + NS * DUM
ZPT = ACC_ROWS // NS
WPT = HALF // NS
E_PAD64 = 819200
EPT64 = E_PAD64 // NS
CH64 = 6400
NCH64 = EPT64 // CH64
NSUB = CH64 // 128

_MESH = plsc.VectorSubcoreMesh(core_axis_name="c", subcore_axis_name="s")

_SC_SCRATCH = [
    pltpu.VMEM((N_PAD,), jnp.float32),
    pltpu.VMEM((N_PAD,), jnp.float32),
    pltpu.VMEM((BUF,), jnp.int32),
    pltpu.VMEM((BUF,), jnp.int32),
    pltpu.VMEM((BUF,), jnp.int32),
    pltpu.VMEM((BUF,), jnp.int32),
    pltpu.SemaphoreType.DMA,
    pltpu.SemaphoreType.DMA,
    pltpu.SemaphoreType.DMA,
]


def _sc_pass_body(with_gather, sc_hbm, gt_hbm, g_hbm, out_hbm,
                  g_loc, t_loc, sb0, gb0, sb1, gb1, gsem, sem0, sem1):
    wid = lax.axis_index("s") * NC + lax.axis_index("c")
    base = wid * EPT
    sbufs, gbufs, sems = (sb0, sb1), (gb0, gb1), (sem0, sem1)

    if with_gather:
        gh = pltpu.async_copy(g_hbm, g_loc, gsem)

    def start(c):
        b = c % 2
        off = base + c * CHUNK
        h1 = pltpu.async_copy(sc_hbm.at[pl.ds(off, CHUNK)],
                              sbufs[b].at[pl.ds(0, CHUNK)], sems[b])
        h2 = pltpu.async_copy(gt_hbm.at[pl.ds(off, CHUNK)],
                              gbufs[b].at[pl.ds(0, CHUNK)], sems[b])
        return (h1, h2)

    hs = [None] * NCHUNK
    hs[0] = start(0)

    zv = jnp.zeros((LANES,), jnp.float32)
    ziv = jnp.zeros((LANES,), jnp.int32)
    for b in range(2):
        sbufs[b][pl.ds(NVEC * LANES, LANES)] = ziv
        gbufs[b][pl.ds(NVEC * LANES, LANES)] = ziv

    def zbody(i, carry):
        for k in range(8):
            t_loc[pl.ds(i * 128 + k * LANES, LANES)] = zv
        return carry
    lax.fori_loop(0, N_PAD // 128, zbody, 0)

    if with_gather:
        gh.wait()
    ones = jnp.ones((LANES,), jnp.float32)
    tail_mask = lax.iota(jnp.int32, LANES) < TAIL

    for c in range(NCHUNK):
        b = c % 2
        hs[c][0].wait()
        hs[c][1].wait()
        if c + 1 < NCHUNK:
            hs[c + 1] = start(c + 1)
        sbuf, gbuf = sbufs[b], gbufs[b]

        def body(i, carry):
            for k in range(UNROLL):
                off = i * (UNROLL * LANES) + k * LANES
                svec = sbuf[pl.ds(off, LANES)]
                if with_gather:
                    gvec = gbuf[pl.ds(off, LANES)]
                    vals = plsc.load_gather(g_loc, [gvec])
                else:
                    vals = ones
                plsc.addupdate_scatter(t_loc, [svec], vals)
            return carry
        lax.fori_loop(0, NVEC // UNROLL, body, 0)

        off = NVEC * LANES
        svec = sbuf[pl.ds(off, LANES)]
        if with_gather:
            gvec = gbuf[pl.ds(off, LANES)]
            vals = plsc.load_gather(g_loc, [gvec], mask=tail_mask)
        else:
            vals = ones
        plsc.addupdate_scatter(t_loc, [svec], vals, mask=tail_mask)

    pltpu.sync_copy(t_loc, out_hbm.at[wid])


_sc_spmv = pl.kernel(
    functools.partial(_sc_pass_body, True),
    out_type=jax.ShapeDtypeStruct((NW, N_PAD), jnp.float32),
    mesh=_MESH,
    compiler_params=pltpu.CompilerParams(needs_layout_passes=False),
    scratch_types=_SC_SCRATCH,
)


def _sc_deg_body(dst_hbm, out_hbm, *rest):
    _sc_pass_body(False, dst_hbm, dst_hbm, dst_hbm, out_hbm, *rest)


_sc_deg = pl.kernel(
    _sc_deg_body,
    out_type=jax.ShapeDtypeStruct((NW, N_PAD), jnp.float32),
    mesh=_MESH,
    compiler_params=pltpu.CompilerParams(needs_layout_passes=False),
    scratch_types=_SC_SCRATCH,
)


_SC64_SCRATCH = [
    pltpu.VMEM((CH64,), jnp.int32),
    pltpu.VMEM((CH64,), jnp.int32),
    pltpu.VMEM((2, 128), jnp.int32),
    pltpu.VMEM((128, H), jnp.float32),
    pltpu.VMEM((128, H), jnp.float32),
    pltpu.VMEM_SHARED((ACC_ROWS, H), jnp.float32),
    pltpu.SemaphoreType.DMA,
    pltpu.SemaphoreType.DMA,
    pltpu.SemaphoreType.DMA,
    pltpu.SemaphoreType.DMA,
    pltpu.SemaphoreType.DMA,
]


def _sc64_body(src_hbm, dst_hbm, ms_hbm, zacc_hbm, out_hbm,
               sbuf, dbuf, idx2, rowsA, rowsB, acc, esem, gA, gB, sA, sB):
    cid = lax.axis_index("c")
    sid = lax.axis_index("s")
    nbase = cid * HALF
    pltpu.sync_copy(zacc_hbm.at[pl.ds(sid * ZPT, ZPT)],
                    acc.at[pl.ds(sid * ZPT, ZPT)])
    plsc.subcore_barrier()
    dummy0 = HALF + sid * DUM
    dummyv = dummy0 + lax.iota(jnp.int32, LANES)
    ebase = sid * EPT64

    def chunk_body(c, carry):
        off = ebase + c * CH64
        h1 = pltpu.async_copy(src_hbm.at[pl.ds(off, CH64)], sbuf, esem)
        h2 = pltpu.async_copy(dst_hbm.at[pl.ds(off, CH64)], dbuf, esem)
        h1.wait()
        h2.wait()

        def pair_body(p, carry2):
            q0 = p * 2
            subs = ((q0, 0, rowsA, gA, sA), (q0 + 1, 1, rowsB, gB, sB))
            ghs = []
            for q, j, rows, gsem, _ in subs:
                ghs.append(pltpu.async_copy(
                    ms_hbm.at[sbuf.at[pl.ds(q * 128, 128)]], rows, gsem))
            for q, j, rows, _, _ in subs:
                def prep(k, carry3):
                    dvec = dbuf[pl.ds(q * 128 + k * LANES, LANES)]
                    local = dvec - nbase
                    ok = (local >= 0) & (local < HALF)
                    idx2[j, pl.ds(k * LANES, LANES)] = jnp.where(
                        ok, local, dummyv)
                    return carry3
                lax.fori_loop(0, 128 // LANES, prep, 0)
            shs = []
            for (q, j, rows, _, ssem), gh in zip(subs, ghs):
                gh.wait()
                shs.append(pltpu.async_copy(
                    rows, acc.at[idx2.at[j]], ssem, add=True))
            for sh in shs:
                sh.wait()
            return carry2
        lax.fori_loop(0, NSUB // 2, pair_body, 0)
        return carry
    lax.fori_loop(0, NCH64, chunk_body, 0)

    plsc.subcore_barrier()
    pltpu.sync_copy(acc.at[pl.ds(sid * WPT, WPT)],
                    out_hbm.at[pl.ds(nbase + sid * WPT, WPT)])


_sc64 = pl.kernel(
    _sc64_body,
    out_type=jax.ShapeDtypeStruct((N_PAD, H), jnp.float32),
    mesh=_MESH,
    compiler_params=pltpu.CompilerParams(needs_layout_passes=False,
                                         use_tc_tiling_on_sc=False),
    scratch_types=_SC64_SCRATCH,
)


def _r(a):
    return a.astype(jnp.bfloat16).astype(jnp.float32)


def _dinv2_body(parts_ref, x_ref, dinv_ref, gx_ref):
    deg = jnp.sum(parts_ref[...], axis=0) + 1.0
    dinv = lax.rsqrt(deg)
    dinv = dinv * (1.5 - 0.5 * deg * dinv * dinv)
    dinv = dinv * (1.5 - 0.5 * deg * dinv * dinv)
    idx = (lax.broadcasted_iota(jnp.int32, (ROWS, 128), 0) * 128
           + lax.broadcasted_iota(jnp.int32, (ROWS, 128), 1))
    dinv = jnp.where(idx < N, dinv, 0.0)
    dinv_ref[...] = dinv
    gx_ref[...] = dinv * x_ref[...]


def _merge_body(parts_ref, dinv_ref, gprev_ref, y_ref):
    t = jnp.sum(parts_ref[...], axis=0)
    y_ref[...] = dinv_ref[...] * (t + gprev_ref[...])


def _big1_body(y_ref, dinv_ref, W1T_ref, b1T_ref, W2T_ref, ms_ref):
    c1 = W1T_ref[...] * y_ref[...] + b1T_ref[...]
    M = jnp.dot(_r(W2T_ref[...]), _r(c1),
                precision=lax.Precision.HIGHEST)
    ms_ref[...] = dinv_ref[...] * M


def _big2_body(T_ref, ms_ref, dinv_ref, u_ref, b2T_ref, q_ref):
    c2 = dinv_ref[...] * (T_ref[...] + ms_ref[...]) + b2T_ref[...]
    q = jnp.sum(_r(c2) * u_ref[...], axis=1)
    q_ref[...] = q.reshape(1, H)


def _final_body(q_ref, W3_ref, b3_ref, l1W_ref, l1b_ref, l2W_ref, l2b_ref,
                out_ref):
    dot = functools.partial(jnp.dot, precision=lax.Precision.HIGHEST)
    pooled = dot(q_ref[...], _r(W3_ref[...])) + jnp.float32(N) * b3_ref[...]
    h = jnp.maximum(dot(_r(pooled), _r(l1W_ref[...])) + l1b_ref[...], 0.0)
    out_ref[...] = dot(h, l2W_ref[...]) + l2b_ref[...]


def kernel(x, edge_index, W1, b1, W2, b2, W3, b3,
           lin1_W, lin1_b, lin2_W, lin2_b):
    ei = edge_index.astype(jnp.int32)
    src, dst = ei[0], ei[1]

    deg_parts = _sc_deg(dst)
    x_pad2d = jnp.pad(x[:, 0], (0, N_PAD - N)).reshape(ROWS, 128)
    dinv2d, gx2d = pl.pallas_call(
        _dinv2_body,
        out_shape=(jax.ShapeDtypeStruct((ROWS, 128), jnp.float32),
                   jax.ShapeDtypeStruct((ROWS, 128), jnp.float32)),
    )(deg_parts.reshape(NW, ROWS, 128), x_pad2d)
    dinv = dinv2d.reshape(N_PAD)

    merge = pl.pallas_call(
        _merge_body,
        out_shape=jax.ShapeDtypeStruct((ROWS, 128), jnp.float32),
    )
    tu = _sc_spmv(src, dst, dinv)
    u2d = merge(tu.reshape(NW, ROWS, 128), dinv2d, dinv2d)
    ty = _sc_spmv(dst, src, gx2d.reshape(N_PAD))
    y2d = merge(ty.reshape(NW, ROWS, 128), dinv2d, gx2d)

    ms_cm = pl.pallas_call(
        _big1_body,
        out_shape=jax.ShapeDtypeStruct((H, N_PAD), jnp.float32),
    )(y2d.reshape(1, N_PAD), dinv2d.reshape(1, N_PAD),
      W1.reshape(H, 1), b1.reshape(H, 1), W2.T)

    ms_rm = ms_cm.T
    padv = N + (jnp.arange(E_PAD64 - E, dtype=jnp.int32) % (N_PAD - N))
    srcp = jnp.concatenate([src, padv])
    dstp = jnp.concatenate([dst, padv])
    zacc = jnp.zeros((ACC_ROWS, H), jnp.float32)
    T_rm = _sc64(srcp, dstp, ms_rm, zacc)

    q = pl.pallas_call(
        _big2_body,
        out_shape=jax.ShapeDtypeStruct((1, H), jnp.float32),
    )(T_rm.T, ms_cm, dinv2d.reshape(1, N_PAD), u2d.reshape(1, N_PAD),
      b2.reshape(H, 1))

    out = pl.pallas_call(
        _final_body,
        out_shape=jax.ShapeDtypeStruct((1, 1), jnp.float32),
    )(q, W3, b3.reshape(1, H), lin1_W, lin1_b.reshape(1, H),
      lin2_W, lin2_b.reshape(1, 1))
    return out
